# baseline (speedup 1.0000x reference)
import functools

import jax
import jax.numpy as jnp
from jax import lax
from jax.experimental import pallas as pl
from jax.experimental.pallas import tpu as pltpu
from jax.experimental.pallas import tpu_sc as plsc

_NUM_BUCKETS = 1000000
_HASH_DIM = 64
_MODEL_DIM = 1024
_MULT = 92821

_NC = 2
_NS = 16
_NW = _NC * _NS
_L = 16

_TOK = 32768
_CHUNK = _TOK // _NW
_GBLK = 128
_NG = _CHUNK // _GBLK


def _sc_gather(ids_hbm, prev_hbm, table_hbm, emb_hbm,
               ids_v, prev_v, h_v, rows_v, sem):
    wid = lax.axis_index("s") * _NC + lax.axis_index("c")
    base = wid * _CHUNK
    pltpu.sync_copy(ids_hbm.at[pl.ds(base, _CHUNK)], ids_v)
    pltpu.sync_copy(prev_hbm.at[pl.ds(base, _CHUNK)], prev_v)
    for i in range(_CHUNK // _L):
        a = prev_v[pl.ds(i * _L, _L)]
        b = ids_v[pl.ds(i * _L, _L)]
        t = a * _MULT + b
        r = lax.rem(t, _NUM_BUCKETS)
        h = jnp.where(r < 0, r + _NUM_BUCKETS, r)
        h_v[i // (_GBLK // _L), pl.ds((i % (_GBLK // _L)) * _L, _L)] = h
    descs = []
    for j in range(_NG):
        descs.append(
            pltpu.async_copy(table_hbm.at[h_v.at[j]],
                             rows_v.at[pl.ds(j * _GBLK, _GBLK)], sem))
    for d in descs:
        d.wait()
    pltpu.sync_copy(rows_v, emb_hbm.at[pl.ds(base, _CHUNK)])


_gather_call = functools.partial(
    pl.kernel,
    out_type=jax.ShapeDtypeStruct((_TOK, _HASH_DIM), jnp.float32),
    mesh=plsc.VectorSubcoreMesh(
        core_axis_name="c", subcore_axis_name="s",
        num_cores=_NC, num_subcores=_NS),
    scratch_types=[
        pltpu.VMEM((_CHUNK,), jnp.int32),
        pltpu.VMEM((_CHUNK,), jnp.int32),
        pltpu.VMEM((_NG, _GBLK), jnp.int32),
        pltpu.VMEM((_CHUNK, _HASH_DIM), jnp.float32),
        pltpu.SemaphoreType.DMA,
    ],
    compiler_params=pltpu.CompilerParams(use_tc_tiling_on_sc=False),
)(_sc_gather)


def _mm_body(emb_ref, wt_ref, o_ref):
    o_ref[...] = lax.dot_general(
        emb_ref[...], wt_ref[...], (((1,), (0,)), ((), ())),
        preferred_element_type=jnp.float32)


def kernel(input_ids, table, W):
    ids = input_ids.astype(jnp.int32)
    bsz, seqlen = ids.shape
    prev = jnp.concatenate(
        [jnp.zeros((bsz, 1), dtype=ids.dtype), ids[:, :-1]], axis=1)
    ids_f = ids.reshape(-1)
    prev_f = prev.reshape(-1)

    emb = _gather_call(ids_f, prev_f, table)

    blk = 512
    out = pl.pallas_call(
        _mm_body,
        grid=(_TOK // blk,),
        in_specs=[
            pl.BlockSpec((blk, _HASH_DIM), lambda i: (i, 0)),
            pl.BlockSpec((_HASH_DIM, _MODEL_DIM), lambda i: (0, 0)),
        ],
        out_specs=pl.BlockSpec((blk, _MODEL_DIM), lambda i: (i, 0)),
        out_shape=jax.ShapeDtypeStruct((_TOK, _MODEL_DIM), jnp.float32),
    )(emb, W.T)
    return out.reshape(bsz, seqlen, _MODEL_DIM)
